# baseline (speedup 1.0000x reference)
import functools

import jax
import jax.numpy as jnp
import numpy as np
from jax import lax
from jax.experimental import pallas as pl
from jax.experimental.pallas import tpu as pltpu
from jax.experimental.pallas import tpu_sc as plsc

_K = 11
_K2 = _K * _K
_H = 512
_W = 512
_HW = _H * _W
_B = 2
_N = 16384
_NC = 2
_NS = 16
_L = 16
_NPER = _N // _NS
_PTS = 128
_MB = _NPER // _PTS
_QC = _PTS // _L
_PPT = _HW // _NS
_PCH = 1024
_NCONST = 20
_NPROJ = 11


def _sqrtf(a):
    ai = plsc.bitcast(a, jnp.int32)
    y = plsc.bitcast(jnp.int32(0x5F3759DF) - (ai >> 1), jnp.float32)
    for _ in range(4):
        y = y * (1.5 - 0.5 * a * y * y)
    return a * y


def _bf16q(x):
    i = plsc.bitcast(x, jnp.int32)
    r = i + jnp.int32(0x7FFF) + ((i >> 16) & 1)
    return plsc.bitcast(r & jnp.int32(-65536), jnp.float32)


def _round_i32(x):
    xc = jnp.minimum(jnp.maximum(x, -1024.0), 2048.0)
    t = xc.astype(jnp.int32)
    f = xc - t.astype(jnp.float32)
    af = jnp.abs(f)
    adj = jnp.logical_or(af > 0.5,
                         jnp.logical_and(af == 0.5, (t & 1) == 1))
    sgn = jnp.where(xc >= 0.0, 1, -1)
    return t + jnp.where(adj, sgn, 0)


def _splat_body(gdata_hbm, consts_hbm,
                rgb_hbm, dep_hbm,
                cr, cg, cb, ca, cz,
                parr, cvm, idxbuf, valbuf, cbuf, obuf, zbuf,
                candxi, candxw, candyi, candyw,
                sem0, sem1, sem2, sem3):
    c = lax.axis_index("c")
    s = lax.axis_index("s")
    canvs = (cr, cg, cb, ca, cz)
    zeros = jnp.zeros((_L,), jnp.float32)

    for p in range(9):
        pltpu.sync_copy(
            gdata_hbm.at[pl.ds((c * 9 + p) * _N + s * _NPER, _NPER)],
            parr.at[p, pl.ds(0, _NPER)])
    pltpu.sync_copy(consts_hbm.at[pl.ds(c * (_NCONST * _L), _NCONST * _L)],
                    cvm)

    def zfill(i, _):
        zbuf[pl.ds(i * _L, _L)] = zeros
        return 0

    lax.fori_loop(0, _PCH // _L, zfill, 0)

    def zcan(t, _):
        d = pl.ds(s * _PPT + t * _PCH, _PCH)
        for cv_ in canvs:
            pltpu.sync_copy(zbuf, cv_.at[d])
        return 0

    lax.fori_loop(0, _PPT // _PCH, zcan, 0)

    cv = [cvm[pl.ds(i * _L, _L)] for i in range(_NCONST)]
    (r00, r01, r02, r10, r11, r12, r20, r21, r22,
     t0, t1, t2, k00, k01, k02, k10, k11, k12, fx, fy) = cv

    def proj_body(i, _):
        d = pl.ds(i * _L, _L)
        x = parr[0, d]; y = parr[1, d]; z = parr[2, d]
        sxx = parr[3, d]; syy = parr[4, d]
        r = parr[5, d]; g = parr[6, d]; b = parr[7, d]
        opa = parr[8, d]
        xq = _bf16q(x); yq = _bf16q(y); zq = _bf16q(z)
        xc0 = r00 * xq + r01 * yq + r02 * zq + t0
        xc1 = r10 * xq + r11 * yq + r12 * zq + t1
        xc2 = r20 * xq + r21 * yq + r22 * zq + t2
        zs = jnp.maximum(xc2, 1e-6)
        xn = _bf16q(xc0 / zs)
        yn = _bf16q(xc1 / zs)
        u = k00 * xn + k01 * yn + k02
        v = k10 * xn + k11 * yn + k12
        sx = _sqrtf(jnp.maximum(sxx, 1e-9)) * fx / zs
        sy = _sqrtf(jnp.maximum(syy, 1e-9)) * fy / zs
        parr[0, d] = u
        parr[1, d] = v
        parr[2, d] = sx
        parr[3, d] = sy
        parr[4, d] = 1.0 / jnp.maximum(sx, 1e-6)
        parr[5, d] = 1.0 / jnp.maximum(sy, 1e-6)
        parr[6, d] = opa * r
        parr[7, d] = opa * g
        parr[8, d] = opa * b
        parr[10, d] = opa * xc2
        parr[9, d] = opa
        return 0

    lax.fori_loop(0, _NPER // _L, proj_body, 0)
    plsc.subcore_barrier()

    sems = (sem0, sem1, sem2, sem3)

    def make_cands(m):
        for q in range(_QC):
            d = pl.ds(m * _PTS + q * _L, _L)
            dq = pl.ds(q * _L, _L)
            u = parr[0, d]; v = parr[1, d]
            sx = parr[2, d]; sy = parr[3, d]
            isx = parr[4, d]; isy = parr[5, d]
            for oi in range(_K):
                of = float(oi - _K // 2)
                px = _round_i32(u + of * sx)
                pxc = jnp.minimum(jnp.maximum(px, 0), _W - 1)
                gx = of * isx
                wx = jnp.exp(-0.5 * (gx * gx))
                candxi[oi, dq] = pxc
                candxw[oi, dq] = jnp.where(px == pxc, wx, 0.0)
                py = _round_i32(v + of * sy)
                pyc = jnp.minimum(jnp.maximum(py, 0), _H - 1)
                gy = of * isy
                wy = jnp.exp(-0.5 * (gy * gy))
                candyi[oi, dq] = pyc * _W
                candyw[oi, dq] = jnp.where(py == pyc, wy, 0.0)

    def emit_batch(m, jj, sl):
        oyi = jj // _K
        oxi = jj - oyi * _K
        for q in range(_QC):
            d = pl.ds(m * _PTS + q * _L, _L)
            dq = pl.ds(q * _L, _L)
            aw = candxw[oxi, dq] * candyw[oyi, dq]
            idxbuf[sl, dq] = candyi[oyi, dq] + candxi[oxi, dq]
            for ch in range(5):
                valbuf[sl, ch, dq] = parr[6 + ch, d] * aw
        for ch in range(5):
            pltpu.async_copy(valbuf.at[sl, ch],
                             canvs[ch].at[idxbuf.at[sl]], sems[sl],
                             add=True)

    def drain(sl):
        for ch in range(5):
            pltpu.make_async_copy(valbuf.at[sl, ch],
                                  canvs[ch].at[idxbuf.at[sl]],
                                  sems[sl]).wait()

    def batch_outer(m, _):
        make_cands(m)

        def joff(i, _):
            for sl in range(4):
                t = m * _K2 + i * 4 + sl

                @pl.when(t >= 4)
                def _(sl=sl):
                    drain(sl)
                emit_batch(m, i * 4 + sl, sl)
            return 0

        lax.fori_loop(0, _K2 // 4, joff, 0)
        t = m * _K2 + _K2 - 1

        @pl.when(t >= 4)
        def _():
            drain(0)
        emit_batch(m, _K2 - 1, 0)
        return 0

    lax.fori_loop(0, _MB, batch_outer, 0)
    for sl in range(4):
        drain(sl)
    plsc.subcore_barrier()

    def norm_body(t, _):
        pix0 = s * _PPT + t * _PCH
        for ch in range(5):
            pltpu.sync_copy(canvs[ch].at[pl.ds(pix0, _PCH)], cbuf.at[ch])

        def group_body(gi, _):
            d = pl.ds(gi * _L, _L)
            inv = 1.0 / jnp.maximum(cbuf[3, d], 1e-6)
            obuf[0, d] = cbuf[0, d] * inv
            obuf[1, d] = cbuf[1, d] * inv
            obuf[2, d] = cbuf[2, d] * inv
            obuf[3, d] = cbuf[4, d] * inv
            return 0

        lax.fori_loop(0, _PCH // _L, group_body, 0)
        for ch in range(3):
            pltpu.sync_copy(
                obuf.at[ch],
                rgb_hbm.at[pl.ds((c * 3 + ch) * _HW + pix0, _PCH)])
        pltpu.sync_copy(obuf.at[3],
                        dep_hbm.at[pl.ds(c * _HW + pix0, _PCH)])
        return 0

    lax.fori_loop(0, _PPT // _PCH, norm_body, 0)


@functools.partial(jax.jit, static_argnums=(7, 8))
def _render(pos, cov_diag, rgb, opa, K, R, t, image_h, image_w):
    f32 = jnp.float32
    gdata = jnp.concatenate(
        [jnp.swapaxes(pos, 1, 2),
         jnp.swapaxes(cov_diag[..., 0:2], 1, 2),
         jnp.swapaxes(rgb, 1, 2),
         jnp.swapaxes(opa, 1, 2)], axis=1).astype(f32).reshape(-1)
    bq = lambda a: a.astype(jnp.bfloat16).astype(f32)
    consts = jnp.concatenate(
        [bq(R.reshape(_B, 9)), t.reshape(_B, 3),
         bq(K[:, 0, :].reshape(_B, 3)), bq(K[:, 1, :].reshape(_B, 3)),
         K[:, 0, 0].reshape(_B, 1), K[:, 1, 1].reshape(_B, 1)], axis=1)
    consts = jnp.broadcast_to(
        consts[:, :, None], (_B, _NCONST, _L)).astype(f32).reshape(-1)
    mesh = plsc.VectorSubcoreMesh(core_axis_name="c", subcore_axis_name="s",
                                  num_cores=_NC, num_subcores=_NS)
    rgb_out, dep_out = pl.kernel(
        _splat_body,
        out_type=[jax.ShapeDtypeStruct((_B * 3 * _HW,), f32),
                  jax.ShapeDtypeStruct((_B * _HW,), f32)],
        mesh=mesh,
        scratch_types=[
            pltpu.VMEM_SHARED((_HW,), f32),
            pltpu.VMEM_SHARED((_HW,), f32),
            pltpu.VMEM_SHARED((_HW,), f32),
            pltpu.VMEM_SHARED((_HW,), f32),
            pltpu.VMEM_SHARED((_HW,), f32),
            pltpu.VMEM((_NPROJ, _NPER), f32),
            pltpu.VMEM((_NCONST * _L,), f32),
            pltpu.VMEM((4, _PTS), jnp.int32),
            pltpu.VMEM((4, 5, _PTS), f32),
            pltpu.VMEM((5, _PCH), f32),
            pltpu.VMEM((4, _PCH), f32),
            pltpu.VMEM((_PCH,), f32),
            pltpu.VMEM((_K, _PTS), jnp.int32),
            pltpu.VMEM((_K, _PTS), f32),
            pltpu.VMEM((_K, _PTS), jnp.int32),
            pltpu.VMEM((_K, _PTS), f32),
            pltpu.SemaphoreType.DMA,
            pltpu.SemaphoreType.DMA,
            pltpu.SemaphoreType.DMA,
            pltpu.SemaphoreType.DMA,
        ],
        compiler_params=pltpu.CompilerParams(use_tc_tiling_on_sc=False,
                                             needs_layout_passes=False),
    )(gdata, consts)
    return (rgb_out.reshape(_B, 3, _H, _W),
            dep_out.reshape(_B, 1, _H, _W))


def kernel(pos_bnh3, cov_diag_bnh3, rgb_bnh3, opa_bnh1, K_b33, R_b33, t_b3,
           image_h, image_w):
    return _render(pos_bnh3, cov_diag_bnh3, rgb_bnh3, opa_bnh1, K_b33, R_b33,
                   t_b3, _H, _W)


# speedup vs baseline: 1.0181x; 1.0181x over previous
import functools

import jax
import jax.numpy as jnp
import numpy as np
from jax import lax
from jax.experimental import pallas as pl
from jax.experimental.pallas import tpu as pltpu
from jax.experimental.pallas import tpu_sc as plsc

_K = 11
_K2 = _K * _K
_H = 512
_W = 512
_HW = _H * _W
_B = 2
_N = 16384
_NC = 2
_NS = 16
_L = 16
_NPER = _N // _NS
_PTS = 128
_MB = _NPER // _PTS
_QC = _PTS // _L
_PPT = _HW // _NS
_PCH = 1024
_NCONST = 20
_NPROJ = 11


def _sqrtf(a):
    ai = plsc.bitcast(a, jnp.int32)
    y = plsc.bitcast(jnp.int32(0x5F3759DF) - (ai >> 1), jnp.float32)
    for _ in range(4):
        y = y * (1.5 - 0.5 * a * y * y)
    return a * y


def _bf16q(x):
    i = plsc.bitcast(x, jnp.int32)
    r = i + jnp.int32(0x7FFF) + ((i >> 16) & 1)
    return plsc.bitcast(r & jnp.int32(-65536), jnp.float32)


def _round_i32(x):
    xc = jnp.minimum(jnp.maximum(x, -1024.0), 2048.0)
    t = xc.astype(jnp.int32)
    f = xc - t.astype(jnp.float32)
    af = jnp.abs(f)
    adj = jnp.logical_or(af > 0.5,
                         jnp.logical_and(af == 0.5, (t & 1) == 1))
    sgn = jnp.where(xc >= 0.0, 1, -1)
    return t + jnp.where(adj, sgn, 0)


def _splat_body(gdata_hbm, consts_hbm,
                rgb_hbm, dep_hbm,
                cr, cg, cb, ca, cz,
                parr, cvm, idxbuf, valbuf, cbuf, obuf, zbuf,
                candxi, candxw, candyi, candyw,
                sem0, sem1):
    c = lax.axis_index("c")
    s = lax.axis_index("s")
    canvs = (cr, cg, cb, ca, cz)
    zeros = jnp.zeros((_L,), jnp.float32)

    for p in range(9):
        pltpu.sync_copy(
            gdata_hbm.at[pl.ds((c * 9 + p) * _N + s * _NPER, _NPER)],
            parr.at[p, pl.ds(0, _NPER)])
    pltpu.sync_copy(consts_hbm.at[pl.ds(c * (_NCONST * _L), _NCONST * _L)],
                    cvm)

    def zfill(i, _):
        zbuf[pl.ds(i * _L, _L)] = zeros
        return 0

    lax.fori_loop(0, _PCH // _L, zfill, 0)

    def zcan(t, _):
        d = pl.ds(s * _PPT + t * _PCH, _PCH)
        for cv_ in canvs:
            pltpu.async_copy(zbuf, cv_.at[d], sem1)
        return 0

    lax.fori_loop(0, _PPT // _PCH, zcan, 0)

    def zdrain(t, _):
        d = pl.ds(s * _PPT + t * _PCH, _PCH)
        for cv_ in canvs:
            pltpu.make_async_copy(zbuf, cv_.at[d], sem1).wait()
        return 0

    lax.fori_loop(0, _PPT // _PCH, zdrain, 0)

    cv = [cvm[pl.ds(i * _L, _L)] for i in range(_NCONST)]
    (r00, r01, r02, r10, r11, r12, r20, r21, r22,
     t0, t1, t2, k00, k01, k02, k10, k11, k12, fx, fy) = cv

    def proj_body(i, _):
        d = pl.ds(i * _L, _L)
        x = parr[0, d]; y = parr[1, d]; z = parr[2, d]
        sxx = parr[3, d]; syy = parr[4, d]
        r = parr[5, d]; g = parr[6, d]; b = parr[7, d]
        opa = parr[8, d]
        xq = _bf16q(x); yq = _bf16q(y); zq = _bf16q(z)
        xc0 = r00 * xq + r01 * yq + r02 * zq + t0
        xc1 = r10 * xq + r11 * yq + r12 * zq + t1
        xc2 = r20 * xq + r21 * yq + r22 * zq + t2
        zs = jnp.maximum(xc2, 1e-6)
        xn = _bf16q(xc0 / zs)
        yn = _bf16q(xc1 / zs)
        u = k00 * xn + k01 * yn + k02
        v = k10 * xn + k11 * yn + k12
        sx = _sqrtf(jnp.maximum(sxx, 1e-9)) * fx / zs
        sy = _sqrtf(jnp.maximum(syy, 1e-9)) * fy / zs
        parr[0, d] = u
        parr[1, d] = v
        parr[2, d] = sx
        parr[3, d] = sy
        parr[4, d] = 1.0 / jnp.maximum(sx, 1e-6)
        parr[5, d] = 1.0 / jnp.maximum(sy, 1e-6)
        parr[6, d] = opa * r
        parr[7, d] = opa * g
        parr[8, d] = opa * b
        parr[10, d] = opa * xc2
        parr[9, d] = opa
        return 0

    lax.fori_loop(0, _NPER // _L, proj_body, 0)
    plsc.subcore_barrier()

    sems = (sem0, sem1)

    def make_cands(m):
        for q in range(_QC):
            d = pl.ds(m * _PTS + q * _L, _L)
            dq = pl.ds(q * _L, _L)
            u = parr[0, d]; v = parr[1, d]
            sx = parr[2, d]; sy = parr[3, d]
            isx = parr[4, d]; isy = parr[5, d]
            for oi in range(_K):
                of = float(oi - _K // 2)
                px = _round_i32(u + of * sx)
                pxc = jnp.minimum(jnp.maximum(px, 0), _W - 1)
                gx = of * isx
                wx = jnp.exp(-0.5 * (gx * gx))
                candxi[oi, dq] = pxc
                candxw[oi, dq] = jnp.where(px == pxc, wx, 0.0)
                py = _round_i32(v + of * sy)
                pyc = jnp.minimum(jnp.maximum(py, 0), _H - 1)
                gy = of * isy
                wy = jnp.exp(-0.5 * (gy * gy))
                candyi[oi, dq] = pyc * _W
                candyw[oi, dq] = jnp.where(py == pyc, wy, 0.0)

    def emit_batch(m, jj, sl):
        oyi = jj // _K
        oxi = jj - oyi * _K
        for q in range(_QC):
            d = pl.ds(m * _PTS + q * _L, _L)
            dq = pl.ds(q * _L, _L)
            aw = candxw[oxi, dq] * candyw[oyi, dq]
            idxbuf[sl, dq] = candyi[oyi, dq] + candxi[oxi, dq]
            for ch in range(5):
                valbuf[sl, ch, dq] = parr[6 + ch, d] * aw
        for ch in range(5):
            pltpu.async_copy(valbuf.at[sl, ch],
                             canvs[ch].at[idxbuf.at[sl]], sems[sl],
                             add=True)

    def drain(sl):
        for ch in range(5):
            pltpu.make_async_copy(valbuf.at[sl, ch],
                                  canvs[ch].at[idxbuf.at[sl]],
                                  sems[sl]).wait()

    def batch_outer(m, _):
        make_cands(m)

        def joff(i, _):
            for sl in range(2):
                t = m * _K2 + i * 2 + sl

                @pl.when(t >= 2)
                def _(sl=sl):
                    drain(sl)
                emit_batch(m, i * 2 + sl, sl)
            return 0

        lax.fori_loop(0, _K2 // 2, joff, 0)
        t = m * _K2 + _K2 - 1

        @pl.when(t >= 2)
        def _():
            drain(0)
        emit_batch(m, _K2 - 1, 0)
        return 0

    lax.fori_loop(0, _MB, batch_outer, 0)
    drain(0)
    drain(1)
    plsc.subcore_barrier()

    def norm_body(t, _):
        pix0 = s * _PPT + t * _PCH
        for ch in range(5):
            pltpu.sync_copy(canvs[ch].at[pl.ds(pix0, _PCH)], cbuf.at[ch])

        def group_body(gi, _):
            d = pl.ds(gi * _L, _L)
            inv = 1.0 / jnp.maximum(cbuf[3, d], 1e-6)
            obuf[0, d] = cbuf[0, d] * inv
            obuf[1, d] = cbuf[1, d] * inv
            obuf[2, d] = cbuf[2, d] * inv
            obuf[3, d] = cbuf[4, d] * inv
            return 0

        lax.fori_loop(0, _PCH // _L, group_body, 0)
        for ch in range(3):
            pltpu.sync_copy(
                obuf.at[ch],
                rgb_hbm.at[pl.ds((c * 3 + ch) * _HW + pix0, _PCH)])
        pltpu.sync_copy(obuf.at[3],
                        dep_hbm.at[pl.ds(c * _HW + pix0, _PCH)])
        return 0

    lax.fori_loop(0, _PPT // _PCH, norm_body, 0)


@functools.partial(jax.jit, static_argnums=(7, 8))
def _render(pos, cov_diag, rgb, opa, K, R, t, image_h, image_w):
    f32 = jnp.float32
    gdata = jnp.concatenate(
        [jnp.swapaxes(pos, 1, 2),
         jnp.swapaxes(cov_diag[..., 0:2], 1, 2),
         jnp.swapaxes(rgb, 1, 2),
         jnp.swapaxes(opa, 1, 2)], axis=1).astype(f32).reshape(-1)
    bq = lambda a: a.astype(jnp.bfloat16).astype(f32)
    consts = jnp.concatenate(
        [bq(R.reshape(_B, 9)), t.reshape(_B, 3),
         bq(K[:, 0, :].reshape(_B, 3)), bq(K[:, 1, :].reshape(_B, 3)),
         K[:, 0, 0].reshape(_B, 1), K[:, 1, 1].reshape(_B, 1)], axis=1)
    consts = jnp.broadcast_to(
        consts[:, :, None], (_B, _NCONST, _L)).astype(f32).reshape(-1)
    mesh = plsc.VectorSubcoreMesh(core_axis_name="c", subcore_axis_name="s",
                                  num_cores=_NC, num_subcores=_NS)
    rgb_out, dep_out = pl.kernel(
        _splat_body,
        out_type=[jax.ShapeDtypeStruct((_B * 3 * _HW,), f32),
                  jax.ShapeDtypeStruct((_B * _HW,), f32)],
        mesh=mesh,
        scratch_types=[
            pltpu.VMEM_SHARED((_HW,), f32),
            pltpu.VMEM_SHARED((_HW,), f32),
            pltpu.VMEM_SHARED((_HW,), f32),
            pltpu.VMEM_SHARED((_HW,), f32),
            pltpu.VMEM_SHARED((_HW,), f32),
            pltpu.VMEM((_NPROJ, _NPER), f32),
            pltpu.VMEM((_NCONST * _L,), f32),
            pltpu.VMEM((2, _PTS), jnp.int32),
            pltpu.VMEM((2, 5, _PTS), f32),
            pltpu.VMEM((5, _PCH), f32),
            pltpu.VMEM((4, _PCH), f32),
            pltpu.VMEM((_PCH,), f32),
            pltpu.VMEM((_K, _PTS), jnp.int32),
            pltpu.VMEM((_K, _PTS), f32),
            pltpu.VMEM((_K, _PTS), jnp.int32),
            pltpu.VMEM((_K, _PTS), f32),
            pltpu.SemaphoreType.DMA,
            pltpu.SemaphoreType.DMA,
        ],
        compiler_params=pltpu.CompilerParams(use_tc_tiling_on_sc=False,
                                             needs_layout_passes=False),
    )(gdata, consts)
    return (rgb_out.reshape(_B, 3, _H, _W),
            dep_out.reshape(_B, 1, _H, _W))


def kernel(pos_bnh3, cov_diag_bnh3, rgb_bnh3, opa_bnh1, K_b33, R_b33, t_b3,
           image_h, image_w):
    return _render(pos_bnh3, cov_diag_bnh3, rgb_bnh3, opa_bnh1, K_b33, R_b33,
                   t_b3, _H, _W)
